# baseline (speedup 1.0000x reference)
import functools

import jax
import jax.numpy as jnp
from jax import lax
from jax.experimental import pallas as pl
from jax.experimental.pallas import tpu as pltpu
from jax.experimental.pallas import tpu_sc as plsc

B = 16384
K = 32
NW = 32
BPW = B // NW
NCHUNK = 4
CH = BPW // NCHUNK

@functools.cache
def _get_sc_gather():
    mesh = plsc.VectorSubcoreMesh(core_axis_name="c", subcore_axis_name="s")

    @functools.partial(
        pl.kernel,
        out_type=(
            jax.ShapeDtypeStruct((B, K), jnp.float32),
            jax.ShapeDtypeStruct((B, K), jnp.float32),
        ),
        mesh=mesh,
        scratch_types=[
            pltpu.VMEM((NCHUNK, CH), jnp.int32),
            pltpu.VMEM((NCHUNK, CH), jnp.int32),
            pltpu.VMEM((BPW, K), jnp.float32),
            pltpu.VMEM((BPW, K), jnp.float32),
            pltpu.SemaphoreType.DMA,
        ],
        compiler_params=pltpu.CompilerParams(use_tc_tiling_on_sc=False),
    )
    def _sc_gather(idx_u_hbm, idx_i_hbm, user_hbm, item_hbm, ue_out, ie_out,
                   idxu_v, idxi_v, ue_v, ie_v, sem):
        wid = lax.axis_index("s") * 2 + lax.axis_index("c")
        base = wid * BPW
        pltpu.sync_copy(idx_u_hbm.at[wid], idxu_v)
        pltpu.sync_copy(idx_i_hbm.at[wid], idxi_v)
        copies = []
        for j in range(NCHUNK):
            copies.append(pltpu.async_copy(
                user_hbm.at[idxu_v.at[j]], ue_v.at[pl.ds(j * CH, CH)], sem))
            copies.append(pltpu.async_copy(
                item_hbm.at[idxi_v.at[j]], ie_v.at[pl.ds(j * CH, CH)], sem))
        for c in copies:
            c.wait()
        pltpu.sync_copy(ue_v, ue_out.at[pl.ds(base, BPW)])
        pltpu.sync_copy(ie_v, ie_out.at[pl.ds(base, BPW)])

    return _sc_gather


BR = 2048


def _mlp_body(ue_ref, ie_ref, wj1_ref, bj1_ref, wj2_ref, wu1_ref, bu1_ref,
              wu2_ref, wi1_ref, bi1_ref, wi2_ref, o1_ref, o2_ref, o3_ref):
    ue = ue_ref[...]
    ie = ie_ref[...]
    dn = (((1,), (1,)), ((), ()))
    wj1 = wj1_ref[...]
    hj = jnp.maximum(
        lax.dot_general(ue, wj1[:, :K], dn, preferred_element_type=jnp.float32)
        + lax.dot_general(ie, wj1[:, K:], dn, preferred_element_type=jnp.float32)
        + bj1_ref[...], 0.0)
    joint = lax.dot_general(hj, wj2_ref[...], dn,
                            preferred_element_type=jnp.float32)
    hu = jnp.maximum(
        lax.dot_general(ue, wu1_ref[...], dn, preferred_element_type=jnp.float32)
        + bu1_ref[...], 0.0)
    uo = lax.dot_general(hu, wu2_ref[...], dn,
                         preferred_element_type=jnp.float32)
    hi = jnp.maximum(
        lax.dot_general(ie, wi1_ref[...], dn, preferred_element_type=jnp.float32)
        + bi1_ref[...], 0.0)
    io = lax.dot_general(hi, wi2_ref[...], dn,
                         preferred_element_type=jnp.float32)
    o1_ref[...] = joint + uo + io
    o2_ref[...] = joint + uo
    o3_ref[...] = io


def _row_block(i):
    return (i, 0)


def _whole(i):
    return (0, 0)


_mlp_call = pl.pallas_call(
    _mlp_body,
    grid=(B // BR,),
    in_specs=[
        pl.BlockSpec((BR, K), _row_block),
        pl.BlockSpec((BR, K), _row_block),
        pl.BlockSpec((K, 2 * K), _whole),
        pl.BlockSpec((1, K), _whole),
        pl.BlockSpec((1, K), _whole),
        pl.BlockSpec((K, K), _whole),
        pl.BlockSpec((1, K), _whole),
        pl.BlockSpec((1, K), _whole),
        pl.BlockSpec((K, K), _whole),
        pl.BlockSpec((1, K), _whole),
        pl.BlockSpec((1, K), _whole),
    ],
    out_specs=[
        pl.BlockSpec((BR, 1), _row_block),
        pl.BlockSpec((BR, 1), _row_block),
        pl.BlockSpec((BR, 1), _row_block),
    ],
    out_shape=[
        jax.ShapeDtypeStruct((B, 1), jnp.float32),
        jax.ShapeDtypeStruct((B, 1), jnp.float32),
        jax.ShapeDtypeStruct((B, 1), jnp.float32),
    ],
)


def kernel(x, user_emb, item_emb, Wj1, bj1, Wj2, Wu1, bu1, Wu2, Wi1, bi1, Wi2):
    xi = x.astype(jnp.int32)
    idx_u = xi[:, 0].reshape(NW, NCHUNK, CH)
    idx_i = xi[:, 1].reshape(NW, NCHUNK, CH)
    ue, ie = _get_sc_gather()(idx_u, idx_i, user_emb, item_emb)
    o1, o2, o3 = _mlp_call(
        ue, ie, Wj1, bj1.reshape(1, K), Wj2, Wu1, bu1.reshape(1, K), Wu2,
        Wi1, bi1.reshape(1, K), Wi2)
    return (o1, o2, o3)


# speedup vs baseline: 1.6180x; 1.6180x over previous
import functools

import jax
import jax.numpy as jnp
from jax import lax
from jax.experimental import pallas as pl
from jax.experimental.pallas import tpu as pltpu
from jax.experimental.pallas import tpu_sc as plsc

B = 16384
V = 1000000
K = 32
NW = 32
BPW = B // NW
NCHUNK = 4
CH = BPW // NCHUNK
PACK = 4
VP = V // PACK

PC = 8192
PB = PC // PACK
NPB = -(-V // PC)
VPAD = NPB * PB


def _pack_body(ut_ref, it_ref, pu_ref, pi_ref):
    rows = lax.broadcasted_iota(jnp.int32, (K, K), 0)
    cols = lax.broadcasted_iota(jnp.int32, (K, K), 1)
    eye = jnp.where(rows == cols, 1.0, 0.0).astype(jnp.float32)
    dn = (((0,), (0,)), ((), ()))

    def packed(x_ref):
        xt = lax.dot_general(x_ref[...], eye, dn,
                             preferred_element_type=jnp.float32)
        return jnp.concatenate(
            [xt[j * PB:(j + 1) * PB, :] for j in range(PACK)], axis=1)

    pu_ref[...] = packed(ut_ref)
    pi_ref[...] = packed(it_ref)


_pack_call = pl.pallas_call(
    _pack_body,
    grid=(NPB,),
    in_specs=[
        pl.BlockSpec((K, PC), lambda i: (0, i)),
        pl.BlockSpec((K, PC), lambda i: (0, i)),
    ],
    out_specs=[
        pl.BlockSpec((PB, PACK * K), lambda i: (i, 0)),
        pl.BlockSpec((PB, PACK * K), lambda i: (i, 0)),
    ],
    out_shape=[
        jax.ShapeDtypeStruct((VPAD, PACK * K), jnp.float32),
        jax.ShapeDtypeStruct((VPAD, PACK * K), jnp.float32),
    ],
)


@functools.cache
def _get_sc_gather():
    mesh = plsc.VectorSubcoreMesh(core_axis_name="c", subcore_axis_name="s")

    @functools.partial(
        pl.kernel,
        out_type=(
            jax.ShapeDtypeStruct((B, PACK * K), jnp.float32),
            jax.ShapeDtypeStruct((B, PACK * K), jnp.float32),
        ),
        mesh=mesh,
        scratch_types=[
            pltpu.VMEM((NCHUNK, CH), jnp.int32),
            pltpu.VMEM((NCHUNK, CH), jnp.int32),
            pltpu.VMEM((BPW, PACK * K), jnp.float32),
            pltpu.SemaphoreType.DMA,
        ],
    )
    def _sc_gather(idx_u_hbm, idx_i_hbm, pu_hbm, pi_hbm, gu_out, gi_out,
                   idx_v, gidx_v, rows_v, sem):
        wid = lax.axis_index("s") * 2 + lax.axis_index("c")
        base = wid * BPW
        for idx_hbm, tab_hbm, out_hbm in (
                (idx_u_hbm, pu_hbm, gu_out), (idx_i_hbm, pi_hbm, gi_out)):
            pltpu.sync_copy(idx_hbm.at[wid], idx_v)
            for c in range(NCHUNK):
                for v in range(CH // 16):
                    sl = pl.ds(v * 16, 16)
                    i = idx_v.at[c][sl]
                    gidx_v.at[c][sl] = (
                        lax.shift_left(lax.shift_right_logical(i, 13), 11)
                        | (i & 2047))
            copies = [
                pltpu.async_copy(tab_hbm.at[gidx_v.at[c]],
                                 rows_v.at[pl.ds(c * CH, CH)], sem)
                for c in range(NCHUNK)
            ]
            for cp in copies:
                cp.wait()
            pltpu.sync_copy(rows_v, out_hbm.at[pl.ds(base, BPW)])

    return _sc_gather


BR = 2048


def _mlp_body(gu_ref, gi_ref, iu_ref, ii_ref, wj1_ref, bj1_ref, wj2_ref,
              wu1_ref, bu1_ref, wu2_ref, wi1_ref, bi1_ref, wi2_ref,
              o1_ref, o2_ref, o3_ref):
    f32 = jnp.float32

    def pick(g_ref, i_ref):
        p = lax.shift_right_logical(i_ref[...], 11) & 3
        g = g_ref[...]
        e = jnp.where(p == 1, g[:, K:2 * K], g[:, :K])
        e = jnp.where(p == 2, g[:, 2 * K:3 * K], e)
        return jnp.where(p == 3, g[:, 3 * K:], e)

    ue = pick(gu_ref, iu_ref)
    ie = pick(gi_ref, ii_ref)
    dn = (((1,), (1,)), ((), ()))
    wj1 = wj1_ref[...]
    hj = jnp.maximum(
        lax.dot_general(ue, wj1[:, :K], dn, preferred_element_type=f32)
        + lax.dot_general(ie, wj1[:, K:], dn, preferred_element_type=f32)
        + bj1_ref[...], 0.0)
    joint = lax.dot_general(hj, wj2_ref[...], dn, preferred_element_type=f32)
    hu = jnp.maximum(
        lax.dot_general(ue, wu1_ref[...], dn, preferred_element_type=f32)
        + bu1_ref[...], 0.0)
    uo = lax.dot_general(hu, wu2_ref[...], dn, preferred_element_type=f32)
    hi = jnp.maximum(
        lax.dot_general(ie, wi1_ref[...], dn, preferred_element_type=f32)
        + bi1_ref[...], 0.0)
    io = lax.dot_general(hi, wi2_ref[...], dn, preferred_element_type=f32)
    o1_ref[...] = joint + uo + io
    o2_ref[...] = joint + uo
    o3_ref[...] = io


def _row_block(i):
    return (i, 0)


def _whole(i):
    return (0, 0)


_mlp_call = pl.pallas_call(
    _mlp_body,
    grid=(B // BR,),
    in_specs=[
        pl.BlockSpec((BR, PACK * K), _row_block),
        pl.BlockSpec((BR, PACK * K), _row_block),
        pl.BlockSpec((BR, 1), _row_block),
        pl.BlockSpec((BR, 1), _row_block),
        pl.BlockSpec((K, 2 * K), _whole),
        pl.BlockSpec((1, K), _whole),
        pl.BlockSpec((1, K), _whole),
        pl.BlockSpec((K, K), _whole),
        pl.BlockSpec((1, K), _whole),
        pl.BlockSpec((1, K), _whole),
        pl.BlockSpec((K, K), _whole),
        pl.BlockSpec((1, K), _whole),
        pl.BlockSpec((1, K), _whole),
    ],
    out_specs=[
        pl.BlockSpec((BR, 1), _row_block),
        pl.BlockSpec((BR, 1), _row_block),
        pl.BlockSpec((BR, 1), _row_block),
    ],
    out_shape=[
        jax.ShapeDtypeStruct((B, 1), jnp.float32),
        jax.ShapeDtypeStruct((B, 1), jnp.float32),
        jax.ShapeDtypeStruct((B, 1), jnp.float32),
    ],
)


def kernel(x, user_emb, item_emb, Wj1, bj1, Wj2, Wu1, bu1, Wu2, Wi1, bi1, Wi2):
    xi = x.astype(jnp.int32)
    idx_u = xi[:, 0]
    idx_i = xi[:, 1]
    pu, pi = _pack_call(user_emb.T, item_emb.T)
    gu, gi = _get_sc_gather()(idx_u.reshape(NW, NCHUNK, CH),
                              idx_i.reshape(NW, NCHUNK, CH), pu, pi)
    o1, o2, o3 = _mlp_call(
        gu, gi, idx_u.reshape(B, 1), idx_i.reshape(B, 1),
        Wj1, bj1.reshape(1, K), Wj2, Wu1, bu1.reshape(1, K), Wu2,
        Wi1, bi1.reshape(1, K), Wi2)
    return (o1, o2, o3)


# speedup vs baseline: 3.1951x; 1.9747x over previous
import functools

import jax
import jax.numpy as jnp
from jax import lax
from jax.experimental import pallas as pl
from jax.experimental.pallas import tpu as pltpu
from jax.experimental.pallas import tpu_sc as plsc

B = 16384
V = 1000000
K = 32
NW = 32
BPW = B // NW
NCHUNK = 4
CH = BPW // NCHUNK
PACK = 4
VP = V // PACK

PC = 8192
PB = PC // PACK
NPB = -(-V // PC)
VPAD = NPB * PB


def _pack_body(ut_ref, it_ref, pu_ref, pi_ref):
    def packed(x_ref):
        x = x_ref[...]
        x4 = jnp.concatenate(
            [x[:, j * PB:(j + 1) * PB] for j in range(PACK)], axis=0)
        return x4.T

    pu_ref[...] = packed(ut_ref)
    pi_ref[...] = packed(it_ref)


_pack_call = pl.pallas_call(
    _pack_body,
    grid=(NPB,),
    in_specs=[
        pl.BlockSpec((K, PC), lambda i: (0, i)),
        pl.BlockSpec((K, PC), lambda i: (0, i)),
    ],
    out_specs=[
        pl.BlockSpec((PB, PACK * K), lambda i: (i, 0)),
        pl.BlockSpec((PB, PACK * K), lambda i: (i, 0)),
    ],
    out_shape=[
        jax.ShapeDtypeStruct((VPAD, PACK * K), jnp.float32),
        jax.ShapeDtypeStruct((VPAD, PACK * K), jnp.float32),
    ],
)


@functools.cache
def _get_sc_gather():
    mesh = plsc.VectorSubcoreMesh(core_axis_name="c", subcore_axis_name="s")

    @functools.partial(
        pl.kernel,
        out_type=(
            jax.ShapeDtypeStruct((B, PACK * K), jnp.float32),
            jax.ShapeDtypeStruct((B, PACK * K), jnp.float32),
        ),
        mesh=mesh,
        scratch_types=[
            pltpu.VMEM((NCHUNK, CH), jnp.int32),
            pltpu.VMEM((NCHUNK, CH), jnp.int32),
            pltpu.VMEM((BPW, PACK * K), jnp.float32),
            pltpu.SemaphoreType.DMA,
        ],
    )
    def _sc_gather(idx_u_hbm, idx_i_hbm, pu_hbm, pi_hbm, gu_out, gi_out,
                   idx_v, gidx_v, rows_v, sem):
        wid = lax.axis_index("s") * 2 + lax.axis_index("c")
        base = wid * BPW
        for idx_hbm, tab_hbm, out_hbm in (
                (idx_u_hbm, pu_hbm, gu_out), (idx_i_hbm, pi_hbm, gi_out)):
            pltpu.sync_copy(idx_hbm.at[wid], idx_v)
            for c in range(NCHUNK):
                for v in range(CH // 16):
                    sl = pl.ds(v * 16, 16)
                    i = idx_v.at[c][sl]
                    gidx_v.at[c][sl] = (
                        lax.shift_left(lax.shift_right_logical(i, 13), 11)
                        | (i & 2047))
            copies = [
                pltpu.async_copy(tab_hbm.at[gidx_v.at[c]],
                                 rows_v.at[pl.ds(c * CH, CH)], sem)
                for c in range(NCHUNK)
            ]
            for cp in copies:
                cp.wait()
            pltpu.sync_copy(rows_v, out_hbm.at[pl.ds(base, BPW)])

    return _sc_gather


BR = 2048


def _mlp_body(gu_ref, gi_ref, iu_ref, ii_ref, wj1_ref, bj1_ref, wj2_ref,
              wu1_ref, bu1_ref, wu2_ref, wi1_ref, bi1_ref, wi2_ref,
              o1_ref, o2_ref, o3_ref):
    f32 = jnp.float32

    def pick(g_ref, i_ref):
        p = lax.shift_right_logical(i_ref[...], 11) & 3
        g = g_ref[...]
        e = jnp.where(p == 1, g[:, K:2 * K], g[:, :K])
        e = jnp.where(p == 2, g[:, 2 * K:3 * K], e)
        return jnp.where(p == 3, g[:, 3 * K:], e)

    ue = pick(gu_ref, iu_ref)
    ie = pick(gi_ref, ii_ref)
    dn = (((1,), (1,)), ((), ()))
    wj1 = wj1_ref[...]
    hj = jnp.maximum(
        lax.dot_general(ue, wj1[:, :K], dn, preferred_element_type=f32)
        + lax.dot_general(ie, wj1[:, K:], dn, preferred_element_type=f32)
        + bj1_ref[...], 0.0)
    joint = lax.dot_general(hj, wj2_ref[...], dn, preferred_element_type=f32)
    hu = jnp.maximum(
        lax.dot_general(ue, wu1_ref[...], dn, preferred_element_type=f32)
        + bu1_ref[...], 0.0)
    uo = lax.dot_general(hu, wu2_ref[...], dn, preferred_element_type=f32)
    hi = jnp.maximum(
        lax.dot_general(ie, wi1_ref[...], dn, preferred_element_type=f32)
        + bi1_ref[...], 0.0)
    io = lax.dot_general(hi, wi2_ref[...], dn, preferred_element_type=f32)
    o1_ref[...] = joint + uo + io
    o2_ref[...] = joint + uo
    o3_ref[...] = io


def _row_block(i):
    return (i, 0)


def _whole(i):
    return (0, 0)


_mlp_call = pl.pallas_call(
    _mlp_body,
    grid=(B // BR,),
    in_specs=[
        pl.BlockSpec((BR, PACK * K), _row_block),
        pl.BlockSpec((BR, PACK * K), _row_block),
        pl.BlockSpec((BR, 1), _row_block),
        pl.BlockSpec((BR, 1), _row_block),
        pl.BlockSpec((K, 2 * K), _whole),
        pl.BlockSpec((1, K), _whole),
        pl.BlockSpec((1, K), _whole),
        pl.BlockSpec((K, K), _whole),
        pl.BlockSpec((1, K), _whole),
        pl.BlockSpec((1, K), _whole),
        pl.BlockSpec((K, K), _whole),
        pl.BlockSpec((1, K), _whole),
        pl.BlockSpec((1, K), _whole),
    ],
    out_specs=[
        pl.BlockSpec((BR, 1), _row_block),
        pl.BlockSpec((BR, 1), _row_block),
        pl.BlockSpec((BR, 1), _row_block),
    ],
    out_shape=[
        jax.ShapeDtypeStruct((B, 1), jnp.float32),
        jax.ShapeDtypeStruct((B, 1), jnp.float32),
        jax.ShapeDtypeStruct((B, 1), jnp.float32),
    ],
)


def kernel(x, user_emb, item_emb, Wj1, bj1, Wj2, Wu1, bu1, Wu2, Wi1, bi1, Wi2):
    xi = x.astype(jnp.int32)
    idx_u = xi[:, 0]
    idx_i = xi[:, 1]
    pu, pi = _pack_call(user_emb.T, item_emb.T)
    gu, gi = _get_sc_gather()(idx_u.reshape(NW, NCHUNK, CH),
                              idx_i.reshape(NW, NCHUNK, CH), pu, pi)
    o1, o2, o3 = _mlp_call(
        gu, gi, idx_u.reshape(B, 1), idx_i.reshape(B, 1),
        Wj1, bj1.reshape(1, K), Wj2, Wu1, bu1.reshape(1, K), Wu2,
        Wi1, bi1.reshape(1, K), Wi2)
    return (o1, o2, o3)


# speedup vs baseline: 3.8366x; 1.2008x over previous
import functools

import jax
import jax.numpy as jnp
from jax import lax
from jax.experimental import pallas as pl
from jax.experimental.pallas import tpu as pltpu
from jax.experimental.pallas import tpu_sc as plsc

B = 16384
V = 1000000
K = 32
NW = 32
BPW = B // NW
NCHUNK = 4
CH = BPW // NCHUNK
PACK = 4

PC = 16384
PB = PC // PACK
LG_PC = 14
LG_PB = 12
NPB = -(-V // PC)
VPAD = NPB * PB


def _pack_body(ut_ref, it_ref, pu_ref, pi_ref):
    def packed(x_ref):
        x = x_ref[...]
        x4 = jnp.concatenate(
            [x[:, j * PB:(j + 1) * PB] for j in range(PACK)], axis=0)
        return x4.T

    pu_ref[...] = packed(ut_ref)
    pi_ref[...] = packed(it_ref)


_pack_call = pl.pallas_call(
    _pack_body,
    grid=(NPB,),
    in_specs=[
        pl.BlockSpec((K, PC), lambda i: (0, i)),
        pl.BlockSpec((K, PC), lambda i: (0, i)),
    ],
    out_specs=[
        pl.BlockSpec((PB, PACK * K), lambda i: (i, 0)),
        pl.BlockSpec((PB, PACK * K), lambda i: (i, 0)),
    ],
    out_shape=[
        jax.ShapeDtypeStruct((VPAD, PACK * K), jnp.float32),
        jax.ShapeDtypeStruct((VPAD, PACK * K), jnp.float32),
    ],
)


@functools.cache
def _get_sc_gather():
    mesh = plsc.VectorSubcoreMesh(core_axis_name="c", subcore_axis_name="s")

    @functools.partial(
        pl.kernel,
        out_type=(
            jax.ShapeDtypeStruct((B, K), jnp.float32),
            jax.ShapeDtypeStruct((B, K), jnp.float32),
        ),
        mesh=mesh,
        scratch_types=[
            pltpu.VMEM((NCHUNK, CH), jnp.int32),
            pltpu.VMEM((NCHUNK, CH), jnp.int32),
            pltpu.VMEM((BPW,), jnp.int32),
            pltpu.VMEM((2, CH, PACK * K), jnp.float32),
            pltpu.VMEM((BPW, K), jnp.float32),
            pltpu.SemaphoreType.DMA,
        ],
        compiler_params=pltpu.CompilerParams(needs_layout_passes=False),
    )
    def _sc_gather(idx_u_hbm, idx_i_hbm, pu_hbm, pi_hbm, gu_out, gi_out,
                   idx_v, gidx_v, off_v, rows_v, comp_v, sem):
        wid = lax.axis_index("s") * 2 + lax.axis_index("c")
        base = wid * BPW
        lane = lax.iota(jnp.int32, 16)
        for idx_hbm, tab_hbm, out_hbm in (
                (idx_u_hbm, pu_hbm, gu_out), (idx_i_hbm, pi_hbm, gi_out)):
            pltpu.sync_copy(idx_hbm.at[wid], idx_v)
            for c in range(NCHUNK):
                for v in range(CH // 16):
                    sl = pl.ds(v * 16, 16)
                    i = idx_v.at[c][sl]
                    gidx_v.at[c][sl] = (
                        lax.shift_left(
                            lax.shift_right_logical(i, LG_PC), LG_PB)
                        | (i & (PB - 1)))
                    off_v[pl.ds(c * CH + v * 16, 16)] = lax.shift_left(
                        lax.shift_right_logical(i, LG_PB) & (PACK - 1), 5)
            cps = [None, None]
            cps[0] = pltpu.async_copy(
                tab_hbm.at[gidx_v.at[0]], rows_v.at[0], sem)
            for c in range(NCHUNK):
                if c + 1 < NCHUNK:
                    cps[(c + 1) % 2] = pltpu.async_copy(
                        tab_hbm.at[gidx_v.at[c + 1]],
                        rows_v.at[(c + 1) % 2], sem)
                cps[c % 2].wait()
                buf = rows_v.at[c % 2]

                def compact_row(r, carry):
                    r16 = jnp.full((16,), r, jnp.int32)
                    g16 = r16 + c * CH
                    off = plsc.load_gather(off_v, [g16])
                    for h in range(K // 16):
                        vals = plsc.load_gather(
                            buf, [r16, off + lane + h * 16])
                        plsc.store_scatter(
                            comp_v, [g16, lane + h * 16], vals)
                    return carry

                lax.fori_loop(0, CH, compact_row, 0)
            pltpu.sync_copy(comp_v, out_hbm.at[pl.ds(base, BPW)])

    return _sc_gather


BR = 4096


def _mlp_body(gu_ref, gi_ref, wc_ref, bc_ref, w3_ref,
              o1_ref, o2_ref, o3_ref):
    f32 = jnp.float32
    z = jnp.concatenate([gu_ref[...], gi_ref[...]], axis=1)
    dn = (((1,), (1,)), ((), ()))
    h = jnp.maximum(
        lax.dot_general(z, wc_ref[...], dn, preferred_element_type=f32)
        + bc_ref[...], 0.0)
    s = lax.dot_general(h, w3_ref[...], dn, preferred_element_type=f32)
    ju = s[:, 0:1] + s[:, 1:2]
    io = s[:, 2:3]
    o1_ref[...] = ju + io
    o2_ref[...] = ju
    o3_ref[...] = io


def _row_block(i):
    return (i, 0)


_mlp_call = pl.pallas_call(
    _mlp_body,
    grid=(B // BR,),
    in_specs=[
        pl.BlockSpec((BR, K), _row_block),
        pl.BlockSpec((BR, K), _row_block),
        pl.BlockSpec((3 * K, 2 * K), lambda i: (0, 0)),
        pl.BlockSpec((1, 3 * K), lambda i: (0, 0)),
        pl.BlockSpec((3, 3 * K), lambda i: (0, 0)),
    ],
    out_specs=[
        pl.BlockSpec((BR, 1), _row_block),
        pl.BlockSpec((BR, 1), _row_block),
        pl.BlockSpec((BR, 1), _row_block),
    ],
    out_shape=[
        jax.ShapeDtypeStruct((B, 1), jnp.float32),
        jax.ShapeDtypeStruct((B, 1), jnp.float32),
        jax.ShapeDtypeStruct((B, 1), jnp.float32),
    ],
)


def kernel(x, user_emb, item_emb, Wj1, bj1, Wj2, Wu1, bu1, Wu2, Wi1, bi1, Wi2):
    xi = x.astype(jnp.int32)
    idx_u = xi[:, 0]
    idx_i = xi[:, 1]
    zkk = jnp.zeros((K, K), jnp.float32)
    wc = jnp.concatenate([
        Wj1,
        jnp.concatenate([Wu1, zkk], axis=1),
        jnp.concatenate([zkk, Wi1], axis=1),
    ], axis=0)
    bc = jnp.concatenate([bj1, bu1, bi1]).reshape(1, 3 * K)
    zk1 = jnp.zeros((1, K), jnp.float32)
    w3 = jnp.concatenate([
        jnp.concatenate([Wj2, zk1, zk1], axis=1),
        jnp.concatenate([zk1, Wu2, zk1], axis=1),
        jnp.concatenate([zk1, zk1, Wi2], axis=1),
    ], axis=0)
    pu, pi = _pack_call(user_emb.T, item_emb.T)
    gu, gi = _get_sc_gather()(idx_u.reshape(NW, NCHUNK, CH),
                              idx_i.reshape(NW, NCHUNK, CH), pu, pi)
    o1, o2, o3 = _mlp_call(gu, gi, wc, bc, w3)
    return (o1, o2, o3)


# speedup vs baseline: 4.2213x; 1.1003x over previous
import functools

import jax
import jax.numpy as jnp
from jax import lax
from jax.experimental import pallas as pl
from jax.experimental.pallas import tpu as pltpu
from jax.experimental.pallas import tpu_sc as plsc

B = 16384
V = 1000000
K = 32
NW = 32
BPW = B // NW
NCHUNK = 4
CH = BPW // NCHUNK
PACK = 4

PC = 16384
PB = PC // PACK
LG_PC = 14
LG_PB = 12
NPB = -(-V // PC)
VPAD = NPB * PB


def _pack_body(ut_ref, it_ref, pu_ref, pi_ref):
    def packed(x_ref):
        x = x_ref[...]
        x4 = jnp.concatenate(
            [x[:, j * PB:(j + 1) * PB] for j in range(PACK)], axis=0)
        return x4.T

    pu_ref[...] = packed(ut_ref)
    pi_ref[...] = packed(it_ref)


_pack_call = pl.pallas_call(
    _pack_body,
    grid=(NPB,),
    in_specs=[
        pl.BlockSpec((K, PC), lambda i: (0, i)),
        pl.BlockSpec((K, PC), lambda i: (0, i)),
    ],
    out_specs=[
        pl.BlockSpec((PB, PACK * K), lambda i: (i, 0)),
        pl.BlockSpec((PB, PACK * K), lambda i: (i, 0)),
    ],
    out_shape=[
        jax.ShapeDtypeStruct((VPAD, PACK * K), jnp.float32),
        jax.ShapeDtypeStruct((VPAD, PACK * K), jnp.float32),
    ],
)


@functools.cache
def _get_sc_gather():
    mesh = plsc.VectorSubcoreMesh(core_axis_name="c", subcore_axis_name="s")

    @functools.partial(
        pl.kernel,
        out_type=(
            jax.ShapeDtypeStruct((K, B), jnp.float32),
            jax.ShapeDtypeStruct((K, B), jnp.float32),
        ),
        mesh=mesh,
        scratch_types=[
            pltpu.VMEM((NCHUNK, CH), jnp.int32),
            pltpu.VMEM((NCHUNK, CH), jnp.int32),
            pltpu.VMEM((BPW,), jnp.int32),
            pltpu.VMEM((2, CH, PACK * K), jnp.float32),
            pltpu.VMEM((K, BPW), jnp.float32),
            pltpu.SemaphoreType.DMA,
        ],
        compiler_params=pltpu.CompilerParams(needs_layout_passes=False),
    )
    def _sc_gather(idx_u_hbm, idx_i_hbm, pu_hbm, pi_hbm, gu_out, gi_out,
                   idx_v, gidx_v, off_v, rows_v, comp_v, sem):
        wid = lax.axis_index("s") * 2 + lax.axis_index("c")
        base = wid * BPW
        lane = lax.iota(jnp.int32, 16)
        for idx_hbm, tab_hbm, out_hbm in (
                (idx_u_hbm, pu_hbm, gu_out), (idx_i_hbm, pi_hbm, gi_out)):
            pltpu.sync_copy(idx_hbm.at[wid], idx_v)
            for c in range(NCHUNK):
                for v in range(CH // 16):
                    sl = pl.ds(v * 16, 16)
                    i = idx_v.at[c][sl]
                    gidx_v.at[c][sl] = (
                        lax.shift_left(
                            lax.shift_right_logical(i, LG_PC), LG_PB)
                        | (i & (PB - 1)))
                    off_v[pl.ds(c * CH + v * 16, 16)] = lax.shift_left(
                        lax.shift_right_logical(i, LG_PB) & (PACK - 1), 5)
            cps = [None, None]
            cps[0] = pltpu.async_copy(
                tab_hbm.at[gidx_v.at[0]], rows_v.at[0], sem)
            for c in range(NCHUNK):
                if c + 1 < NCHUNK:
                    cps[(c + 1) % 2] = pltpu.async_copy(
                        tab_hbm.at[gidx_v.at[c + 1]],
                        rows_v.at[(c + 1) % 2], sem)
                cps[c % 2].wait()
                buf = rows_v.at[c % 2]

                def compact_group(rg, carry):
                    r0 = rg * 16
                    row16 = lane + r0
                    off16 = off_v[pl.ds(c * CH + r0, 16)]
                    for k in range(K):
                        vals = plsc.load_gather(buf, [row16, off16 + k])
                        plsc.store_scatter(
                            comp_v,
                            [jnp.full((16,), k, jnp.int32), row16 + c * CH],
                            vals)
                    return carry

                lax.fori_loop(0, CH // 16, compact_group, 0)
            pltpu.sync_copy(comp_v, out_hbm.at[:, pl.ds(base, BPW)])

    return _sc_gather


BC = 4096


def _mlp_body(gu_ref, gi_ref, wc_ref, bc_ref, w3_ref,
              o1_ref, o2_ref, o3_ref):
    f32 = jnp.float32
    z = jnp.concatenate([gu_ref[...], gi_ref[...]], axis=0)
    dn = (((1,), (0,)), ((), ()))
    h = jnp.maximum(
        lax.dot_general(wc_ref[...], z, dn, preferred_element_type=f32)
        + bc_ref[...], 0.0)
    s = lax.dot_general(w3_ref[...], h, dn, preferred_element_type=f32)
    ju = s[0:1, :] + s[1:2, :]
    io = s[2:3, :]
    o1_ref[...] = ju + io
    o2_ref[...] = ju
    o3_ref[...] = io


def _col_block(i):
    return (0, i)


_mlp_call = pl.pallas_call(
    _mlp_body,
    grid=(B // BC,),
    in_specs=[
        pl.BlockSpec((K, BC), _col_block),
        pl.BlockSpec((K, BC), _col_block),
        pl.BlockSpec((3 * K, 2 * K), lambda i: (0, 0)),
        pl.BlockSpec((3 * K, 1), lambda i: (0, 0)),
        pl.BlockSpec((3, 3 * K), lambda i: (0, 0)),
    ],
    out_specs=[
        pl.BlockSpec((1, BC), _col_block),
        pl.BlockSpec((1, BC), _col_block),
        pl.BlockSpec((1, BC), _col_block),
    ],
    out_shape=[
        jax.ShapeDtypeStruct((1, B), jnp.float32),
        jax.ShapeDtypeStruct((1, B), jnp.float32),
        jax.ShapeDtypeStruct((1, B), jnp.float32),
    ],
)


def kernel(x, user_emb, item_emb, Wj1, bj1, Wj2, Wu1, bu1, Wu2, Wi1, bi1, Wi2):
    xi = x.astype(jnp.int32)
    idx_u = xi[:, 0]
    idx_i = xi[:, 1]
    zkk = jnp.zeros((K, K), jnp.float32)
    wc = jnp.concatenate([
        Wj1,
        jnp.concatenate([Wu1, zkk], axis=1),
        jnp.concatenate([zkk, Wi1], axis=1),
    ], axis=0)
    bc = jnp.concatenate([bj1, bu1, bi1]).reshape(3 * K, 1)
    zk1 = jnp.zeros((1, K), jnp.float32)
    w3 = jnp.concatenate([
        jnp.concatenate([Wj2, zk1, zk1], axis=1),
        jnp.concatenate([zk1, Wu2, zk1], axis=1),
        jnp.concatenate([zk1, zk1, Wi2], axis=1),
    ], axis=0)
    pu, pi = _pack_call(user_emb.T, item_emb.T)
    gu, gi = _get_sc_gather()(idx_u.reshape(NW, NCHUNK, CH),
                              idx_i.reshape(NW, NCHUNK, CH), pu, pi)
    o1, o2, o3 = _mlp_call(gu, gi, wc, bc, w3)
    return (o1.reshape(B, 1), o2.reshape(B, 1), o3.reshape(B, 1))


# speedup vs baseline: 4.4413x; 1.0521x over previous
import functools

import jax
import jax.numpy as jnp
from jax import lax
from jax.experimental import pallas as pl
from jax.experimental.pallas import tpu as pltpu
from jax.experimental.pallas import tpu_sc as plsc

B = 16384
V = 1000000
K = 32
NW = 32
BPW = B // NW
NCHUNK = 4
CH = BPW // NCHUNK
PACK = 4

PC = 16384
PB = PC // PACK
LG_PC = 14
LG_PB = 12
NPB = -(-V // PC)
VPAD = NPB * PB


def _pack_body(ut_ref, it_ref, pu_ref, pi_ref):
    def packed(x_ref):
        x = x_ref[...]
        x4 = jnp.concatenate(
            [x[:, j * PB:(j + 1) * PB] for j in range(PACK)], axis=0)
        return x4.T

    pu_ref[...] = packed(ut_ref)
    pi_ref[...] = packed(it_ref)


_pack_call = pl.pallas_call(
    _pack_body,
    grid=(NPB,),
    in_specs=[
        pl.BlockSpec((K, PC), lambda i: (0, i)),
        pl.BlockSpec((K, PC), lambda i: (0, i)),
    ],
    out_specs=[
        pl.BlockSpec((PB, PACK * K), lambda i: (i, 0)),
        pl.BlockSpec((PB, PACK * K), lambda i: (i, 0)),
    ],
    out_shape=[
        jax.ShapeDtypeStruct((VPAD, PACK * K), jnp.float32),
        jax.ShapeDtypeStruct((VPAD, PACK * K), jnp.float32),
    ],
)


@functools.cache
def _get_sc_gather():
    mesh = plsc.VectorSubcoreMesh(core_axis_name="c", subcore_axis_name="s")

    @functools.partial(
        pl.kernel,
        out_type=(
            jax.ShapeDtypeStruct((K, B), jnp.float32),
            jax.ShapeDtypeStruct((K, B), jnp.float32),
        ),
        mesh=mesh,
        scratch_types=[
            pltpu.VMEM((NCHUNK, CH), jnp.int32),
            pltpu.VMEM((NCHUNK, CH), jnp.int32),
            pltpu.VMEM((BPW,), jnp.int32),
            pltpu.VMEM((2, CH, PACK * K), jnp.float32),
            pltpu.VMEM((K, BPW), jnp.float32),
            pltpu.SemaphoreType.DMA,
        ],
        compiler_params=pltpu.CompilerParams(needs_layout_passes=False),
    )
    def _sc_gather(idx_u_hbm, idx_i_hbm, pu_hbm, pi_hbm, gu_out, gi_out,
                   idx_v, gidx_v, off_v, rows_v, comp_v, sem):
        wid = lax.axis_index("s") * 2 + lax.axis_index("c")
        base = wid * BPW
        lane = lax.iota(jnp.int32, 16)
        for idx_hbm, tab_hbm, out_hbm in (
                (idx_u_hbm, pu_hbm, gu_out), (idx_i_hbm, pi_hbm, gi_out)):
            pltpu.sync_copy(idx_hbm.at[wid], idx_v)
            for c in range(NCHUNK):
                for v in range(CH // 16):
                    sl = pl.ds(v * 16, 16)
                    i = idx_v.at[c][sl]
                    gidx_v.at[c][sl] = (
                        lax.shift_left(
                            lax.shift_right_logical(i, LG_PC), LG_PB)
                        | (i & (PB - 1)))
                    off_v[pl.ds(c * CH + v * 16, 16)] = lax.shift_left(
                        lax.shift_right_logical(i, LG_PB) & (PACK - 1), 5)
            cps = [None, None]
            cps[0] = pltpu.async_copy(
                tab_hbm.at[gidx_v.at[0]], rows_v.at[0], sem)
            for c in range(NCHUNK):
                if c + 1 < NCHUNK:
                    cps[(c + 1) % 2] = pltpu.async_copy(
                        tab_hbm.at[gidx_v.at[c + 1]],
                        rows_v.at[(c + 1) % 2], sem)
                cps[c % 2].wait()
                buf = rows_v.at[c % 2]

                def compact_group(rg, carry):
                    r0 = rg * 16
                    row16 = lane + r0
                    off16 = off_v[pl.ds(c * CH + r0, 16)]
                    vals = [plsc.load_gather(buf, [row16, off16 + k])
                            for k in range(K)]
                    for k in range(K):
                        plsc.store_scatter(
                            comp_v,
                            [jnp.full((16,), k, jnp.int32), row16 + c * CH],
                            vals[k])
                    return carry

                lax.fori_loop(0, CH // 16, compact_group, 0)
            pltpu.sync_copy(comp_v, out_hbm.at[:, pl.ds(base, BPW)])

    return _sc_gather


BC = 4096


def _mlp_body(gu_ref, gi_ref, wc_ref, bc_ref, w3_ref,
              o1_ref, o2_ref, o3_ref):
    f32 = jnp.float32
    z = jnp.concatenate([gu_ref[...], gi_ref[...]], axis=0)
    dn = (((1,), (0,)), ((), ()))
    h = jnp.maximum(
        lax.dot_general(wc_ref[...], z, dn, preferred_element_type=f32)
        + bc_ref[...], 0.0)
    s = lax.dot_general(w3_ref[...], h, dn, preferred_element_type=f32)
    ju = s[0:1, :] + s[1:2, :]
    io = s[2:3, :]
    o1_ref[...] = ju + io
    o2_ref[...] = ju
    o3_ref[...] = io


def _col_block(i):
    return (0, i)


_mlp_call = pl.pallas_call(
    _mlp_body,
    grid=(B // BC,),
    in_specs=[
        pl.BlockSpec((K, BC), _col_block),
        pl.BlockSpec((K, BC), _col_block),
        pl.BlockSpec((3 * K, 2 * K), lambda i: (0, 0)),
        pl.BlockSpec((3 * K, 1), lambda i: (0, 0)),
        pl.BlockSpec((3, 3 * K), lambda i: (0, 0)),
    ],
    out_specs=[
        pl.BlockSpec((1, BC), _col_block),
        pl.BlockSpec((1, BC), _col_block),
        pl.BlockSpec((1, BC), _col_block),
    ],
    out_shape=[
        jax.ShapeDtypeStruct((1, B), jnp.float32),
        jax.ShapeDtypeStruct((1, B), jnp.float32),
        jax.ShapeDtypeStruct((1, B), jnp.float32),
    ],
)


def kernel(x, user_emb, item_emb, Wj1, bj1, Wj2, Wu1, bu1, Wu2, Wi1, bi1, Wi2):
    xi = x.astype(jnp.int32)
    idx_u = xi[:, 0]
    idx_i = xi[:, 1]
    zkk = jnp.zeros((K, K), jnp.float32)
    wc = jnp.concatenate([
        Wj1,
        jnp.concatenate([Wu1, zkk], axis=1),
        jnp.concatenate([zkk, Wi1], axis=1),
    ], axis=0)
    bc = jnp.concatenate([bj1, bu1, bi1]).reshape(3 * K, 1)
    zk1 = jnp.zeros((1, K), jnp.float32)
    w3 = jnp.concatenate([
        jnp.concatenate([Wj2, zk1, zk1], axis=1),
        jnp.concatenate([zk1, Wu2, zk1], axis=1),
        jnp.concatenate([zk1, zk1, Wi2], axis=1),
    ], axis=0)
    pu, pi = _pack_call(user_emb.T, item_emb.T)
    gu, gi = _get_sc_gather()(idx_u.reshape(NW, NCHUNK, CH),
                              idx_i.reshape(NW, NCHUNK, CH), pu, pi)
    o1, o2, o3 = _mlp_call(gu, gi, wc, bc, w3)
    return (o1.reshape(B, 1), o2.reshape(B, 1), o3.reshape(B, 1))


# speedup vs baseline: 5.7536x; 1.2955x over previous
import functools

import jax
import jax.numpy as jnp
from jax import lax
from jax.experimental import pallas as pl
from jax.experimental.pallas import tpu as pltpu
from jax.experimental.pallas import tpu_sc as plsc

B = 16384
V = 1000000
K = 32
NW = 32
BPW = B // NW
NCHUNK = 4
CH = BPW // NCHUNK
PACK = 4

PC = 16384
NG = 8
PB = PC // NG
WPR = K // 2
LG_PC = 14
LG_PB = 11
NPB = -(-V // PC)
VPAD = NPB * PB


def _pack_body(ut_ref, it_ref, pu_ref, pi_ref):
    def rnd(u):
        return lax.shift_right_logical(
            u + 0x7FFF + (lax.shift_right_logical(u, 16) & 1), 16)

    def packed(x_ref):
        x = x_ref[...]
        u = lax.bitcast_convert_type(x, jnp.int32)
        w = (rnd(u[:WPR, :])
             | lax.shift_left(rnd(u[WPR:, :]), 16))
        x8 = jnp.concatenate(
            [w[:, j * PB:(j + 1) * PB] for j in range(NG)], axis=0)
        return x8.T

    pu_ref[...] = packed(ut_ref)
    pi_ref[...] = packed(it_ref)


_pack_call = pl.pallas_call(
    _pack_body,
    grid=(NPB,),
    in_specs=[
        pl.BlockSpec((K, PC), lambda i: (0, i)),
        pl.BlockSpec((K, PC), lambda i: (0, i)),
    ],
    out_specs=[
        pl.BlockSpec((PB, NG * WPR), lambda i: (i, 0)),
        pl.BlockSpec((PB, NG * WPR), lambda i: (i, 0)),
    ],
    out_shape=[
        jax.ShapeDtypeStruct((VPAD, NG * WPR), jnp.int32),
        jax.ShapeDtypeStruct((VPAD, NG * WPR), jnp.int32),
    ],
)


@functools.cache
def _get_sc_gather():
    mesh = plsc.VectorSubcoreMesh(core_axis_name="c", subcore_axis_name="s")

    @functools.partial(
        pl.kernel,
        out_type=(
            jax.ShapeDtypeStruct((WPR, B), jnp.int32),
            jax.ShapeDtypeStruct((WPR, B), jnp.int32),
        ),
        mesh=mesh,
        scratch_types=[
            pltpu.VMEM((NCHUNK, CH), jnp.int32),
            pltpu.VMEM((NCHUNK, CH), jnp.int32),
            pltpu.VMEM((BPW,), jnp.int32),
            pltpu.VMEM((2, CH, NG * WPR), jnp.int32),
            pltpu.VMEM((WPR, BPW), jnp.int32),
            pltpu.SemaphoreType.DMA,
        ],
        compiler_params=pltpu.CompilerParams(needs_layout_passes=False),
    )
    def _sc_gather(idx_u_hbm, idx_i_hbm, pu_hbm, pi_hbm, gu_out, gi_out,
                   idx_v, gidx_v, off_v, rows_v, comp_v, sem):
        wid = lax.axis_index("s") * 2 + lax.axis_index("c")
        base = wid * BPW
        lane = lax.iota(jnp.int32, 16)
        for idx_hbm, tab_hbm, out_hbm in (
                (idx_u_hbm, pu_hbm, gu_out), (idx_i_hbm, pi_hbm, gi_out)):
            pltpu.sync_copy(idx_hbm.at[wid], idx_v)
            for c in range(NCHUNK):
                for v in range(CH // 16):
                    sl = pl.ds(v * 16, 16)
                    i = idx_v.at[c][sl]
                    gidx_v.at[c][sl] = (
                        lax.shift_left(
                            lax.shift_right_logical(i, LG_PC), LG_PB)
                        | (i & (PB - 1)))
                    off_v[pl.ds(c * CH + v * 16, 16)] = lax.shift_left(
                        lax.shift_right_logical(i, LG_PB) & (NG - 1), 4)
            cps = [None, None]
            cps[0] = pltpu.async_copy(
                tab_hbm.at[gidx_v.at[0]], rows_v.at[0], sem)
            for c in range(NCHUNK):
                if c + 1 < NCHUNK:
                    cps[(c + 1) % 2] = pltpu.async_copy(
                        tab_hbm.at[gidx_v.at[c + 1]],
                        rows_v.at[(c + 1) % 2], sem)
                cps[c % 2].wait()
                buf = rows_v.at[c % 2]

                def compact_group(rg, carry):
                    r0 = rg * 16
                    row16 = lane + r0
                    off16 = off_v[pl.ds(c * CH + r0, 16)]
                    vals = [plsc.load_gather(buf, [row16, off16 + k])
                            for k in range(WPR)]
                    for k in range(WPR):
                        plsc.store_scatter(
                            comp_v,
                            [jnp.full((16,), k, jnp.int32), row16 + c * CH],
                            vals[k])
                    return carry

                lax.fori_loop(0, CH // 16, compact_group, 0)
            pltpu.sync_copy(comp_v, out_hbm.at[:, pl.ds(base, BPW)])

    return _sc_gather


BC = 4096


def _mlp_body(gu_ref, gi_ref, wc_ref, bc_ref, w3_ref,
              o1_ref, o2_ref, o3_ref):
    f32 = jnp.float32

    def unpack(g_ref):
        g = g_ref[...]
        lo = lax.bitcast_convert_type(lax.shift_left(g, 16), f32)
        hi = lax.bitcast_convert_type(g & jnp.int32(-65536), f32)
        return lo, hi

    ulo, uhi = unpack(gu_ref)
    ilo, ihi = unpack(gi_ref)
    z = jnp.concatenate([ulo, uhi, ilo, ihi], axis=0)
    dn = (((1,), (0,)), ((), ()))
    h = jnp.maximum(
        lax.dot_general(wc_ref[...], z, dn, preferred_element_type=f32)
        + bc_ref[...], 0.0)
    s = lax.dot_general(w3_ref[...], h, dn, preferred_element_type=f32)
    ju = s[0:1, :] + s[1:2, :]
    io = s[2:3, :]
    o1_ref[...] = ju + io
    o2_ref[...] = ju
    o3_ref[...] = io


def _col_block(i):
    return (0, i)


_mlp_call = pl.pallas_call(
    _mlp_body,
    grid=(B // BC,),
    in_specs=[
        pl.BlockSpec((WPR, BC), _col_block),
        pl.BlockSpec((WPR, BC), _col_block),
        pl.BlockSpec((3 * K, 2 * K), lambda i: (0, 0)),
        pl.BlockSpec((3 * K, 1), lambda i: (0, 0)),
        pl.BlockSpec((3, 3 * K), lambda i: (0, 0)),
    ],
    out_specs=[
        pl.BlockSpec((1, BC), _col_block),
        pl.BlockSpec((1, BC), _col_block),
        pl.BlockSpec((1, BC), _col_block),
    ],
    out_shape=[
        jax.ShapeDtypeStruct((1, B), jnp.float32),
        jax.ShapeDtypeStruct((1, B), jnp.float32),
        jax.ShapeDtypeStruct((1, B), jnp.float32),
    ],
)


def kernel(x, user_emb, item_emb, Wj1, bj1, Wj2, Wu1, bu1, Wu2, Wi1, bi1, Wi2):
    xi = x.astype(jnp.int32)
    idx_u = xi[:, 0]
    idx_i = xi[:, 1]
    zkk = jnp.zeros((K, K), jnp.float32)
    wc = jnp.concatenate([
        Wj1,
        jnp.concatenate([Wu1, zkk], axis=1),
        jnp.concatenate([zkk, Wi1], axis=1),
    ], axis=0)
    bc = jnp.concatenate([bj1, bu1, bi1]).reshape(3 * K, 1)
    zk1 = jnp.zeros((1, K), jnp.float32)
    w3 = jnp.concatenate([
        jnp.concatenate([Wj2, zk1, zk1], axis=1),
        jnp.concatenate([zk1, Wu2, zk1], axis=1),
        jnp.concatenate([zk1, zk1, Wi2], axis=1),
    ], axis=0)
    pu, pi = _pack_call(user_emb.T, item_emb.T)
    gu, gi = _get_sc_gather()(idx_u.reshape(NW, NCHUNK, CH),
                              idx_i.reshape(NW, NCHUNK, CH), pu, pi)
    o1, o2, o3 = _mlp_call(gu, gi, wc, bc, w3)
    return (o1.reshape(B, 1), o2.reshape(B, 1), o3.reshape(B, 1))


# speedup vs baseline: 6.0662x; 1.0543x over previous
import functools

import jax
import jax.numpy as jnp
from jax import lax
from jax.experimental import pallas as pl
from jax.experimental.pallas import tpu as pltpu
from jax.experimental.pallas import tpu_sc as plsc

B = 16384
V = 1000000
K = 32
NW = 32
BPW = B // NW
NCHUNK = 4
CH = BPW // NCHUNK
PACK = 4

PC = 32768
NG = 8
PB = PC // NG
WPR = K // 2
LG_PC = 15
LG_PB = 12
NPB = -(-V // PC)
VPAD = NPB * PB


def _pack_body(ut_ref, it_ref, pu_ref, pi_ref):
    def rnd(u):
        return lax.shift_right_logical(
            u + 0x7FFF + (lax.shift_right_logical(u, 16) & 1), 16)

    def packed(x_ref):
        x = x_ref[...]
        u = lax.bitcast_convert_type(x, jnp.int32)
        w = (rnd(u[:WPR, :])
             | lax.shift_left(rnd(u[WPR:, :]), 16))
        x8 = jnp.concatenate(
            [w[:, j * PB:(j + 1) * PB] for j in range(NG)], axis=0)
        return x8.T

    pu_ref[...] = packed(ut_ref)
    pi_ref[...] = packed(it_ref)


_pack_call = pl.pallas_call(
    _pack_body,
    grid=(NPB,),
    in_specs=[
        pl.BlockSpec((K, PC), lambda i: (0, i)),
        pl.BlockSpec((K, PC), lambda i: (0, i)),
    ],
    out_specs=[
        pl.BlockSpec((PB, NG * WPR), lambda i: (i, 0)),
        pl.BlockSpec((PB, NG * WPR), lambda i: (i, 0)),
    ],
    out_shape=[
        jax.ShapeDtypeStruct((VPAD, NG * WPR), jnp.int32),
        jax.ShapeDtypeStruct((VPAD, NG * WPR), jnp.int32),
    ],
)


@functools.cache
def _get_sc_gather():
    mesh = plsc.VectorSubcoreMesh(core_axis_name="c", subcore_axis_name="s")

    @functools.partial(
        pl.kernel,
        out_type=(
            jax.ShapeDtypeStruct((WPR, B), jnp.int32),
            jax.ShapeDtypeStruct((WPR, B), jnp.int32),
        ),
        mesh=mesh,
        scratch_types=[
            pltpu.VMEM((NCHUNK, CH), jnp.int32),
            pltpu.VMEM((NCHUNK, CH), jnp.int32),
            pltpu.VMEM((BPW,), jnp.int32),
            pltpu.VMEM((2, CH, NG * WPR), jnp.int32),
            pltpu.VMEM((WPR, BPW), jnp.int32),
            pltpu.SemaphoreType.DMA,
        ],
        compiler_params=pltpu.CompilerParams(needs_layout_passes=False),
    )
    def _sc_gather(idx_u_hbm, idx_i_hbm, pu_hbm, pi_hbm, gu_out, gi_out,
                   idx_v, gidx_v, off_v, rows_v, comp_v, sem):
        wid = lax.axis_index("s") * 2 + lax.axis_index("c")
        base = wid * BPW
        lane = lax.iota(jnp.int32, 16)
        for idx_hbm, tab_hbm, out_hbm in (
                (idx_u_hbm, pu_hbm, gu_out), (idx_i_hbm, pi_hbm, gi_out)):
            pltpu.sync_copy(idx_hbm.at[wid], idx_v)
            for c in range(NCHUNK):
                for v in range(CH // 16):
                    sl = pl.ds(v * 16, 16)
                    i = idx_v.at[c][sl]
                    gidx_v.at[c][sl] = (
                        lax.shift_left(
                            lax.shift_right_logical(i, LG_PC), LG_PB)
                        | (i & (PB - 1)))
                    off_v[pl.ds(c * CH + v * 16, 16)] = lax.shift_left(
                        lax.shift_right_logical(i, LG_PB) & (NG - 1), 4)
            cps = [None, None]
            cps[0] = pltpu.async_copy(
                tab_hbm.at[gidx_v.at[0]], rows_v.at[0], sem)
            for c in range(NCHUNK):
                if c + 1 < NCHUNK:
                    cps[(c + 1) % 2] = pltpu.async_copy(
                        tab_hbm.at[gidx_v.at[c + 1]],
                        rows_v.at[(c + 1) % 2], sem)
                cps[c % 2].wait()
                buf = rows_v.at[c % 2]

                def compact_group(rg, carry):
                    r0 = rg * 16
                    row16 = lane + r0
                    off16 = off_v[pl.ds(c * CH + r0, 16)]
                    vals = [plsc.load_gather(buf, [row16, off16 + k])
                            for k in range(WPR)]
                    for k in range(WPR):
                        plsc.store_scatter(
                            comp_v,
                            [jnp.full((16,), k, jnp.int32), row16 + c * CH],
                            vals[k])
                    return carry

                lax.fori_loop(0, CH // 16, compact_group, 0)
            pltpu.sync_copy(comp_v, out_hbm.at[:, pl.ds(base, BPW)])

    return _sc_gather


BC = 4096


def _mlp_body(gu_ref, gi_ref, wc_ref, bc_ref, w3_ref,
              o1_ref, o2_ref, o3_ref):
    f32 = jnp.float32

    def unpack(g_ref):
        g = g_ref[...]
        lo = lax.bitcast_convert_type(lax.shift_left(g, 16), f32)
        hi = lax.bitcast_convert_type(g & jnp.int32(-65536), f32)
        return lo, hi

    ulo, uhi = unpack(gu_ref)
    ilo, ihi = unpack(gi_ref)
    z = jnp.concatenate([ulo, uhi, ilo, ihi], axis=0)
    dn = (((1,), (0,)), ((), ()))
    h = jnp.maximum(
        lax.dot_general(wc_ref[...], z, dn, preferred_element_type=f32)
        + bc_ref[...], 0.0)
    s = lax.dot_general(w3_ref[...], h, dn, preferred_element_type=f32)
    ju = s[0:1, :] + s[1:2, :]
    io = s[2:3, :]
    o1_ref[...] = ju + io
    o2_ref[...] = ju
    o3_ref[...] = io


def _col_block(i):
    return (0, i)


_mlp_call = pl.pallas_call(
    _mlp_body,
    grid=(B // BC,),
    in_specs=[
        pl.BlockSpec((WPR, BC), _col_block),
        pl.BlockSpec((WPR, BC), _col_block),
        pl.BlockSpec((3 * K, 2 * K), lambda i: (0, 0)),
        pl.BlockSpec((3 * K, 1), lambda i: (0, 0)),
        pl.BlockSpec((3, 3 * K), lambda i: (0, 0)),
    ],
    out_specs=[
        pl.BlockSpec((1, BC), _col_block),
        pl.BlockSpec((1, BC), _col_block),
        pl.BlockSpec((1, BC), _col_block),
    ],
    out_shape=[
        jax.ShapeDtypeStruct((1, B), jnp.float32),
        jax.ShapeDtypeStruct((1, B), jnp.float32),
        jax.ShapeDtypeStruct((1, B), jnp.float32),
    ],
)


def kernel(x, user_emb, item_emb, Wj1, bj1, Wj2, Wu1, bu1, Wu2, Wi1, bi1, Wi2):
    xi = x.astype(jnp.int32)
    idx_u = xi[:, 0]
    idx_i = xi[:, 1]
    zkk = jnp.zeros((K, K), jnp.float32)
    wc = jnp.concatenate([
        Wj1,
        jnp.concatenate([Wu1, zkk], axis=1),
        jnp.concatenate([zkk, Wi1], axis=1),
    ], axis=0)
    bc = jnp.concatenate([bj1, bu1, bi1]).reshape(3 * K, 1)
    zk1 = jnp.zeros((1, K), jnp.float32)
    w3 = jnp.concatenate([
        jnp.concatenate([Wj2, zk1, zk1], axis=1),
        jnp.concatenate([zk1, Wu2, zk1], axis=1),
        jnp.concatenate([zk1, zk1, Wi2], axis=1),
    ], axis=0)
    pu, pi = _pack_call(user_emb.T, item_emb.T)
    gu, gi = _get_sc_gather()(idx_u.reshape(NW, NCHUNK, CH),
                              idx_i.reshape(NW, NCHUNK, CH), pu, pi)
    o1, o2, o3 = _mlp_call(gu, gi, wc, bc, w3)
    return (o1.reshape(B, 1), o2.reshape(B, 1), o3.reshape(B, 1))
